# baseline (speedup 1.0000x reference)
import functools

import jax
import jax.numpy as jnp
from jax import lax
from jax.experimental import pallas as pl
from jax.experimental.pallas import tpu as pltpu
from jax.experimental.pallas import tpu_sc as plsc

_N = 10000
_E = 320000
_D = 128
_NC = 2
_NS = 16
_NW = _NC * _NS
_EPW = _E // _NW
_CH = 40
_NCHUNK = _EPW // _CH
_RPT = _N // _NS


def _sc_spmm(x, src, dst, w):
    mesh = plsc.VectorSubcoreMesh(core_axis_name="c", subcore_axis_name="s")

    @functools.partial(
        pl.kernel,
        out_type=jax.ShapeDtypeStruct((_NC, _N, _D), jnp.float32),
        mesh=mesh,
        scratch_types=[
            pltpu.VMEM((_NCHUNK, _CH), jnp.int32),
            pltpu.VMEM((_NCHUNK, _CH), jnp.int32),
            pltpu.VMEM((_NCHUNK, _CH), jnp.float32),
            pltpu.VMEM((_CH, _D), jnp.float32),
            pltpu.VMEM((_CH, _D), jnp.float32),
            pltpu.VMEM((_CH, _D), jnp.float32),
            pltpu.VMEM_SHARED((_N, _D), jnp.float32),
            pltpu.SemaphoreType.DMA,
            pltpu.SemaphoreType.DMA,
            pltpu.SemaphoreType.DMA,
            pltpu.SemaphoreType.DMA,
            pltpu.SemaphoreType.DMA,
            pltpu.SemaphoreType.DMA,
        ],
        compiler_params=pltpu.CompilerParams(use_tc_tiling_on_sc=False,
                                             needs_layout_passes=False),
    )
    def k(x_hbm, src_hbm, dst_hbm, w_hbm, out_hbm, sidx, didx, wv,
          rows0, rows1, rows2, acc, gs0, gs1, gs2, ss0, ss1, ss2):
        c = lax.axis_index("c")
        s = lax.axis_index("s")
        wid = s * _NC + c
        rows = (rows0, rows1, rows2)
        gsem = (gs0, gs1, gs2)
        ssem = (ss0, ss1, ss2)

        pltpu.sync_copy(src_hbm.at[wid], sidx)
        pltpu.sync_copy(dst_hbm.at[wid], didx)
        pltpu.sync_copy(w_hbm.at[wid], wv)

        zero16 = jnp.zeros((16,), jnp.float32)

        def zrow(i, carry):
            for kk in range(_D // 16):
                rows0[i, pl.ds(kk * 16, 16)] = zero16
            return carry

        lax.fori_loop(0, _CH, zrow, 0)
        for j in range(_RPT // _CH):
            pltpu.sync_copy(rows0, acc.at[pl.ds(s * _RPT + j * _CH, _CH)])
        tail = _RPT - (_RPT // _CH) * _CH
        if tail:
            pltpu.sync_copy(
                rows0.at[pl.ds(0, tail)],
                acc.at[pl.ds(s * _RPT + (_RPT // _CH) * _CH, tail)])
        plsc.subcore_barrier()

        def start_gather(cix, b):
            pltpu.async_copy(x_hbm.at[sidx.at[cix]], rows[b], gsem[b])

        def wait_gather(b):
            pltpu.make_async_copy(x_hbm.at[sidx.at[0]], rows[b],
                                  gsem[b]).wait()

        def scale(cix, b):
            @plsc.parallel_loop(0, _CH, 1, unroll=2)
            def _scale(e):
                wb = plsc.load_gather(
                    wv, [jnp.full((16,), cix, jnp.int32),
                         jnp.full((16,), e, jnp.int32)])
                for kk in range(_D // 16):
                    sl = pl.ds(kk * 16, 16)
                    rows[b][e, sl] = rows[b][e, sl] * wb

        def start_scatter(cix, b):
            pltpu.async_copy(rows[b], acc.at[didx.at[cix]], ssem[b], add=True)

        def wait_scatter(b):
            pltpu.make_async_copy(rows[b], acc.at[didx.at[0]],
                                  ssem[b]).wait()

        def step(cix, b, first=False, prefetch=True):
            wait_gather(b)
            scale(cix, b)
            start_scatter(cix, b)
            b2 = (b + 2) % 3
            if not first:
                wait_scatter(b2)
            if prefetch:
                start_gather(cix + 2, b2)

        start_gather(0, 0)
        start_gather(1, 1)
        step(0, 0, first=True)
        step(1, 1)
        step(2, 2)

        def body(i, carry):
            c0 = 3 * i
            for jj in range(3):
                b = jj
                cix = c0 + jj
                wait_gather(b)
                scale(cix, b)
                start_scatter(cix, b)
                b2 = (b + 2) % 3
                wait_scatter(b2)

                @pl.when(cix + 2 < _NCHUNK)
                def _():
                    start_gather(cix + 2, b2)
            return carry

        lax.fori_loop(1, _NCHUNK // 3, body, 0)
        step(_NCHUNK - 1, (_NCHUNK - 1) % 3, prefetch=False)
        wait_scatter((_NCHUNK - 1) % 3)
        plsc.subcore_barrier()

        r0 = s * _RPT
        pltpu.sync_copy(acc.at[pl.ds(r0, _RPT)],
                        out_hbm.at[c, pl.ds(r0, _RPT)])

    return k(x, src.reshape(_NW, _NCHUNK, _CH), dst.reshape(_NW, _NCHUNK, _CH),
             w.reshape(_NW, _NCHUNK, _CH))


def _combine_body(scal_ref, p_ref, x_ref, o_ref):
    w0 = scal_ref[0]
    w1 = scal_ref[1]
    b = scal_ref[2]
    o_ref[...] = (p_ref[0] + p_ref[1]) * w0 + x_ref[...] * w1 + b


def _combine(partials, x, scal):
    blk = 1000
    grid = (_N // blk,)
    return pl.pallas_call(
        _combine_body,
        grid=grid,
        in_specs=[
            pl.BlockSpec(memory_space=pltpu.SMEM),
            pl.BlockSpec((_NC, blk, _D), lambda i: (0, i, 0)),
            pl.BlockSpec((blk, _D), lambda i: (i, 0)),
        ],
        out_specs=pl.BlockSpec((blk, _D), lambda i: (i, 0)),
        out_shape=jax.ShapeDtypeStruct((_N, _D), jnp.float32),
    )(scal, partials, x)


def kernel(x, edge_index, edge_weight, conv_w, conv_b):
    dst = edge_index[0]
    src = edge_index[1]
    partials = _sc_spmm(x, src, dst, edge_weight)
    scal = jnp.stack([conv_w[0, 0, 0, 0], conv_w[0, 0, 0, 1], conv_b[0]])
    return _combine(partials, x, scal)


# speedup vs baseline: 1.0822x; 1.0822x over previous
import functools

import jax
import jax.numpy as jnp
import numpy as np
from jax import lax
from jax.experimental import pallas as pl
from jax.experimental.pallas import tpu as pltpu
from jax.experimental.pallas import tpu_sc as plsc

_N = 10000
_E = 320000
_D = 128
_NC = 2
_NS = 16
_NW = _NC * _NS
_EPW = _E // _NW
_CH = 80
_NCHUNK = _EPW // _CH
_RPT = _N // _NS

_PERM = np.empty((_D,), dtype=np.int32)
for _g in range(_D // 32):
    for _i in range(16):
        _PERM[32 * _g + 2 * _i] = 32 * _g + _i
        _PERM[32 * _g + 2 * _i + 1] = 32 * _g + 16 + _i


def _sc_spmm(xp, src, dst, w):
    mesh = plsc.VectorSubcoreMesh(core_axis_name="c", subcore_axis_name="s")

    @functools.partial(
        pl.kernel,
        out_type=jax.ShapeDtypeStruct((_NC, _N, _D), jnp.float32),
        mesh=mesh,
        scratch_types=[
            pltpu.VMEM((_NCHUNK, _CH), jnp.int32),
            pltpu.VMEM((_NCHUNK, _CH), jnp.int32),
            pltpu.VMEM((_NCHUNK, _CH), jnp.float32),
            pltpu.VMEM((_CH, _D), jnp.bfloat16),
            pltpu.VMEM((_CH, _D), jnp.bfloat16),
            pltpu.VMEM((_CH, _D), jnp.float32),
            pltpu.VMEM_SHARED((_N, _D), jnp.float32),
            pltpu.SemaphoreType.DMA,
            pltpu.SemaphoreType.DMA,
        ],
        compiler_params=pltpu.CompilerParams(use_tc_tiling_on_sc=False,
                                             needs_layout_passes=False),
    )
    def k(x_hbm, src_hbm, dst_hbm, w_hbm, out_hbm, sidx, didx, wv,
          rb0, rb1, rf, acc, sem0, sem1):
        c = lax.axis_index("c")
        s = lax.axis_index("s")
        wid = s * _NC + c

        pltpu.sync_copy(src_hbm.at[wid], sidx)
        pltpu.sync_copy(dst_hbm.at[wid], didx)
        pltpu.sync_copy(w_hbm.at[wid], wv)

        zero16 = jnp.zeros((16,), jnp.float32)

        def zrow(i, carry):
            for kk in range(_D // 16):
                rf[i, pl.ds(kk * 16, 16)] = zero16
            return carry

        lax.fori_loop(0, _CH, zrow, 0)
        for j in range(_RPT // _CH):
            pltpu.sync_copy(rf, acc.at[pl.ds(s * _RPT + j * _CH, _CH)])
        tail = _RPT - (_RPT // _CH) * _CH
        if tail:
            pltpu.sync_copy(
                rf.at[pl.ds(0, tail)],
                acc.at[pl.ds(s * _RPT + (_RPT // _CH) * _CH, tail)])
        plsc.subcore_barrier()

        def start_gather(cix, rb, sem):
            pltpu.async_copy(x_hbm.at[sidx.at[cix]], rb, sem)

        def wait_gather(rb, sem):
            pltpu.make_async_copy(x_hbm.at[sidx.at[0]], rb, sem).wait()

        def do_chunk(cix, rb):
            @plsc.parallel_loop(0, _CH, 1, unroll=2)
            def scale(e):
                wb = plsc.load_gather(
                    wv, [jnp.full((16,), cix, jnp.int32),
                         jnp.full((16,), e, jnp.int32)])
                for g in range(_D // 32):
                    v = rb[e, pl.ds(32 * g, 32)]
                    a, b = plsc.unpack(v, format=plsc.PackFormat.INTERLEAVED,
                                       preferred_element_type=jnp.float32)
                    rf[e, pl.ds(32 * g, 16)] = a * wb
                    rf[e, pl.ds(32 * g + 16, 16)] = b * wb

            pltpu.sync_copy(rf, acc.at[didx.at[cix]], add=True)

        start_gather(0, rb0, sem0)

        def pair(i, carry):
            c0 = 2 * i
            start_gather(c0 + 1, rb1, sem1)
            wait_gather(rb0, sem0)
            do_chunk(c0, rb0)
            start_gather(c0 + 2, rb0, sem0)
            wait_gather(rb1, sem1)
            do_chunk(c0 + 1, rb1)
            return carry

        lax.fori_loop(0, (_NCHUNK - 1) // 2, pair, 0)
        wait_gather(rb0, sem0)
        do_chunk(_NCHUNK - 1, rb0)
        plsc.subcore_barrier()

        r0 = s * _RPT
        pltpu.sync_copy(acc.at[pl.ds(r0, _RPT)],
                        out_hbm.at[c, pl.ds(r0, _RPT)])

    return k(xp, src.reshape(_NW, _NCHUNK, _CH),
             dst.reshape(_NW, _NCHUNK, _CH), w.reshape(_NW, _NCHUNK, _CH))


def _combine_body(scal_ref, p_ref, x_ref, o_ref):
    w0 = scal_ref[0]
    w1 = scal_ref[1]
    b = scal_ref[2]
    o_ref[...] = (p_ref[0] + p_ref[1]) * w0 + x_ref[...] * w1 + b


def _combine(partials, x, scal):
    blk = 1000
    grid = (_N // blk,)
    return pl.pallas_call(
        _combine_body,
        grid=grid,
        in_specs=[
            pl.BlockSpec(memory_space=pltpu.SMEM),
            pl.BlockSpec((_NC, blk, _D), lambda i: (0, i, 0)),
            pl.BlockSpec((blk, _D), lambda i: (i, 0)),
        ],
        out_specs=pl.BlockSpec((blk, _D), lambda i: (i, 0)),
        out_shape=jax.ShapeDtypeStruct((_N, _D), jnp.float32),
    )(scal, partials, x)


def kernel(x, edge_index, edge_weight, conv_w, conv_b):
    dst = edge_index[0]
    src = edge_index[1]
    xp = x[:, _PERM].astype(jnp.bfloat16)
    partials = _sc_spmm(xp, src, dst, edge_weight)
    scal = jnp.stack([conv_w[0, 0, 0, 0], conv_w[0, 0, 0, 1], conv_b[0]])
    return _combine(partials, x, scal)


# speedup vs baseline: 1.2729x; 1.1762x over previous
import functools

import jax
import jax.numpy as jnp
from jax import lax
from jax.experimental import pallas as pl
from jax.experimental.pallas import tpu as pltpu
from jax.experimental.pallas import tpu_sc as plsc

_N = 10000
_E = 320000
_D = 128
_NC = 2
_NS = 16
_NW = _NC * _NS
_EPW = _E // _NW
_CH = 80
_NCHUNK = _EPW // _CH
_RPT = _N // _NS


def _sc_spmm(xb, src, dst, w):
    mesh = plsc.VectorSubcoreMesh(core_axis_name="c", subcore_axis_name="s")

    @functools.partial(
        pl.kernel,
        out_type=jax.ShapeDtypeStruct((_NC, _N, _D), jnp.bfloat16),
        mesh=mesh,
        scratch_types=[
            pltpu.VMEM((_NCHUNK, _CH), jnp.int32),
            pltpu.VMEM((_NCHUNK, _CH), jnp.int32),
            pltpu.VMEM((_NCHUNK, _CH), jnp.float32),
            pltpu.VMEM((_CH, _D), jnp.bfloat16),
            pltpu.VMEM((_CH, _D), jnp.bfloat16),
            pltpu.VMEM((_CH, _D), jnp.bfloat16),
            pltpu.VMEM((_CH, _D), jnp.bfloat16),
            pltpu.VMEM_SHARED((_N, _D), jnp.bfloat16),
            pltpu.SemaphoreType.DMA,
            pltpu.SemaphoreType.DMA,
            pltpu.SemaphoreType.DMA,
            pltpu.SemaphoreType.DMA,
        ],
        compiler_params=pltpu.CompilerParams(use_tc_tiling_on_sc=False,
                                             needs_layout_passes=False),
    )
    def k(x_hbm, src_hbm, dst_hbm, w_hbm, out_hbm, sidx, didx, wv,
          rb0, rb1, rf0, rf1, acc, g0, g1, s0, s1):
        c = lax.axis_index("c")
        s = lax.axis_index("s")
        wid = s * _NC + c

        pltpu.sync_copy(src_hbm.at[wid], sidx)
        pltpu.sync_copy(dst_hbm.at[wid], didx)
        pltpu.sync_copy(w_hbm.at[wid], wv)

        zero32 = jnp.zeros((32,), jnp.bfloat16)

        def zrow(i, carry):
            for kk in range(_D // 32):
                rf0[i, pl.ds(kk * 32, 32)] = zero32
            return carry

        lax.fori_loop(0, _CH, zrow, 0)
        for j in range(_RPT // _CH):
            pltpu.sync_copy(rf0, acc.at[pl.ds(s * _RPT + j * _CH, _CH)])
        tail = _RPT - (_RPT // _CH) * _CH
        if tail:
            pltpu.sync_copy(
                rf0.at[pl.ds(0, tail)],
                acc.at[pl.ds(s * _RPT + (_RPT // _CH) * _CH, tail)])
        plsc.subcore_barrier()

        def start_gather(cix, rb, sem):
            pltpu.async_copy(x_hbm.at[sidx.at[cix]], rb, sem)

        def wait_gather(rb, sem):
            pltpu.make_async_copy(x_hbm.at[sidx.at[0]], rb, sem).wait()

        def scale(cix, rb, rf):
            @plsc.parallel_loop(0, _CH, 1, unroll=2)
            def _s(e):
                wb = plsc.load_gather(
                    wv, [jnp.full((16,), cix, jnp.int32),
                         jnp.full((16,), e, jnp.int32)])
                wb2 = plsc.pack(wb, wb, format=plsc.PackFormat.INTERLEAVED)
                for g in range(_D // 32):
                    sl = pl.ds(32 * g, 32)
                    rf[e, sl] = rb[e, sl] * wb2

        def start_scatter(cix, rf, sem):
            pltpu.async_copy(rf, acc.at[didx.at[cix]], sem, add=True)

        def wait_scatter(rf, sem):
            pltpu.make_async_copy(rf, acc.at[didx.at[0]], sem).wait()

        start_gather(0, rb0, g0)
        start_gather(1, rb1, g1)
        wait_gather(rb0, g0)
        scale(0, rb0, rf0)
        start_scatter(0, rf0, s0)
        start_gather(2, rb0, g0)
        wait_gather(rb1, g1)
        scale(1, rb1, rf1)
        start_scatter(1, rf1, s1)

        def pair(i, carry):
            c0 = 2 * i
            start_gather(c0 + 1, rb1, g1)
            wait_gather(rb0, g0)
            wait_scatter(rf0, s0)
            scale(c0, rb0, rf0)
            start_scatter(c0, rf0, s0)
            start_gather(c0 + 2, rb0, g0)
            wait_gather(rb1, g1)
            wait_scatter(rf1, s1)
            scale(c0 + 1, rb1, rf1)
            start_scatter(c0 + 1, rf1, s1)
            return carry

        lax.fori_loop(1, (_NCHUNK - 1) // 2, pair, 0)
        wait_gather(rb0, g0)
        wait_scatter(rf0, s0)
        scale(_NCHUNK - 1, rb0, rf0)
        start_scatter(_NCHUNK - 1, rf0, s0)
        wait_scatter(rf0, s0)
        wait_scatter(rf1, s1)
        plsc.subcore_barrier()

        r0 = s * _RPT
        pltpu.sync_copy(acc.at[pl.ds(r0, _RPT)],
                        out_hbm.at[c, pl.ds(r0, _RPT)])

    return k(xb, src.reshape(_NW, _NCHUNK, _CH),
             dst.reshape(_NW, _NCHUNK, _CH), w.reshape(_NW, _NCHUNK, _CH))


def _combine_body(scal_ref, p_ref, x_ref, o_ref):
    w0 = scal_ref[0]
    w1 = scal_ref[1]
    b = scal_ref[2]
    agg = (p_ref[0].astype(jnp.float32) + p_ref[1].astype(jnp.float32))
    o_ref[...] = agg * w0 + x_ref[...] * w1 + b


def _combine(partials, x, scal):
    blk = 1000
    grid = (_N // blk,)
    return pl.pallas_call(
        _combine_body,
        grid=grid,
        in_specs=[
            pl.BlockSpec(memory_space=pltpu.SMEM),
            pl.BlockSpec((_NC, blk, _D), lambda i: (0, i, 0)),
            pl.BlockSpec((blk, _D), lambda i: (i, 0)),
        ],
        out_specs=pl.BlockSpec((blk, _D), lambda i: (i, 0)),
        out_shape=jax.ShapeDtypeStruct((_N, _D), jnp.float32),
    )(scal, partials, x)


def kernel(x, edge_index, edge_weight, conv_w, conv_b):
    dst = edge_index[0]
    src = edge_index[1]
    xb = x.astype(jnp.bfloat16)
    partials = _sc_spmm(xb, src, dst, edge_weight)
    scal = jnp.stack([conv_w[0, 0, 0, 0], conv_w[0, 0, 0, 1], conv_b[0]])
    return _combine(partials, x, scal)


# speedup vs baseline: 1.4336x; 1.1263x over previous
import functools

import jax
import jax.numpy as jnp
from jax import lax
from jax.experimental import pallas as pl
from jax.experimental.pallas import tpu as pltpu
from jax.experimental.pallas import tpu_sc as plsc

_N = 10000
_E = 320000
_D = 128
_NC = 2
_NS = 16
_NW = _NC * _NS
_EPW = _E // _NW
_CH = 80
_NCHUNK = _EPW // _CH
_RPT = _N // _NS


def _sc_spmm(xb, src, dst, w):
    mesh = plsc.VectorSubcoreMesh(core_axis_name="c", subcore_axis_name="s")

    @functools.partial(
        pl.kernel,
        out_type=jax.ShapeDtypeStruct((_NC, _N, _D), jnp.bfloat16),
        mesh=mesh,
        scratch_types=[
            pltpu.VMEM((_NCHUNK, _CH), jnp.int32),
            pltpu.VMEM((_NCHUNK, _CH), jnp.int32),
            pltpu.VMEM((_NCHUNK, _CH), jnp.float32),
            pltpu.VMEM((_CH, _D), jnp.bfloat16),
            pltpu.VMEM((_CH, _D), jnp.bfloat16),
            pltpu.VMEM((_CH, _D), jnp.bfloat16),
            pltpu.VMEM((_CH, _D), jnp.bfloat16),
            pltpu.VMEM((_CH, _D), jnp.bfloat16),
            pltpu.VMEM((_CH, _D), jnp.bfloat16),
            pltpu.VMEM((_CH, _D), jnp.bfloat16),
            pltpu.VMEM((_CH, _D), jnp.bfloat16),
            pltpu.VMEM_SHARED((_N, _D), jnp.bfloat16),
            pltpu.SemaphoreType.DMA,
            pltpu.SemaphoreType.DMA,
            pltpu.SemaphoreType.DMA,
            pltpu.SemaphoreType.DMA,
            pltpu.SemaphoreType.DMA,
            pltpu.SemaphoreType.DMA,
            pltpu.SemaphoreType.DMA,
            pltpu.SemaphoreType.DMA,
        ],
        compiler_params=pltpu.CompilerParams(use_tc_tiling_on_sc=False,
                                             needs_layout_passes=False),
    )
    def k(x_hbm, src_hbm, dst_hbm, w_hbm, out_hbm, sidx, didx, wv,
          rb0, rb1, rb2, rb3, rf0, rf1, rf2, rf3, acc,
          g0, g1, g2, g3, s0, s1, s2, s3):
        rb = (rb0, rb1, rb2, rb3)
        rf = (rf0, rf1, rf2, rf3)
        gsem = (g0, g1, g2, g3)
        ssem = (s0, s1, s2, s3)
        c = lax.axis_index("c")
        s = lax.axis_index("s")
        wid = s * _NC + c

        pltpu.sync_copy(src_hbm.at[wid], sidx)
        pltpu.sync_copy(dst_hbm.at[wid], didx)
        pltpu.sync_copy(w_hbm.at[wid], wv)

        zero32 = jnp.zeros((32,), jnp.bfloat16)

        def zrow(i, carry):
            for kk in range(_D // 32):
                rf0[i, pl.ds(kk * 32, 32)] = zero32
            return carry

        lax.fori_loop(0, _CH, zrow, 0)
        for j in range(_RPT // _CH):
            pltpu.sync_copy(rf0, acc.at[pl.ds(s * _RPT + j * _CH, _CH)])
        tail = _RPT - (_RPT // _CH) * _CH
        if tail:
            pltpu.sync_copy(
                rf0.at[pl.ds(0, tail)],
                acc.at[pl.ds(s * _RPT + (_RPT // _CH) * _CH, tail)])
        plsc.subcore_barrier()

        def start_gather(cix, b):
            pltpu.async_copy(x_hbm.at[sidx.at[cix]], rb[b], gsem[b])

        def wait_gather(b):
            pltpu.make_async_copy(x_hbm.at[sidx.at[0]], rb[b],
                                  gsem[b]).wait()

        def scale(cix, b):
            @plsc.parallel_loop(0, _CH, 1, unroll=2)
            def _s(e):
                wb = plsc.load_gather(
                    wv, [jnp.full((16,), cix, jnp.int32),
                         jnp.full((16,), e, jnp.int32)])
                wb2 = plsc.pack(wb, wb, format=plsc.PackFormat.INTERLEAVED)
                for g in range(_D // 32):
                    sl = pl.ds(32 * g, 32)
                    rf[b][e, sl] = rb[b][e, sl] * wb2

        def start_scatter(cix, b):
            pltpu.async_copy(rf[b], acc.at[didx.at[cix]], ssem[b], add=True)

        def wait_scatter(b):
            pltpu.make_async_copy(rf[b], acc.at[didx.at[0]],
                                  ssem[b]).wait()

        for b in range(4):
            start_gather(b, b)
        for cc in range(4):
            wait_gather(cc)
            scale(cc, cc)
            start_scatter(cc, cc)
            start_gather(cc + 4, cc)

        def quad(i, carry):
            c0 = 4 * i
            for j in range(4):
                cix = c0 + j
                wait_gather(j)
                wait_scatter(j)
                scale(cix, j)
                start_scatter(cix, j)

                @pl.when(cix + 4 < _NCHUNK)
                def _():
                    start_gather(cix + 4, j)
            return carry

        lax.fori_loop(1, _NCHUNK // 4, quad, 0)
        wait_gather(0)
        wait_scatter(0)
        scale(_NCHUNK - 1, 0)
        start_scatter(_NCHUNK - 1, 0)
        for b in range(4):
            wait_scatter(b)
        plsc.subcore_barrier()

        r0 = s * _RPT
        pltpu.sync_copy(acc.at[pl.ds(r0, _RPT)],
                        out_hbm.at[c, pl.ds(r0, _RPT)])

    return k(xb, src.reshape(_NW, _NCHUNK, _CH),
             dst.reshape(_NW, _NCHUNK, _CH), w.reshape(_NW, _NCHUNK, _CH))


def _combine_body(scal_ref, p_ref, x_ref, o_ref):
    w0 = scal_ref[0]
    w1 = scal_ref[1]
    b = scal_ref[2]
    agg = (p_ref[0].astype(jnp.float32) + p_ref[1].astype(jnp.float32))
    o_ref[...] = agg * w0 + x_ref[...] * w1 + b


def _combine(partials, x, scal):
    blk = 1000
    grid = (_N // blk,)
    return pl.pallas_call(
        _combine_body,
        grid=grid,
        in_specs=[
            pl.BlockSpec(memory_space=pltpu.SMEM),
            pl.BlockSpec((_NC, blk, _D), lambda i: (0, i, 0)),
            pl.BlockSpec((blk, _D), lambda i: (i, 0)),
        ],
        out_specs=pl.BlockSpec((blk, _D), lambda i: (i, 0)),
        out_shape=jax.ShapeDtypeStruct((_N, _D), jnp.float32),
    )(scal, partials, x)


def kernel(x, edge_index, edge_weight, conv_w, conv_b):
    dst = edge_index[0]
    src = edge_index[1]
    xb = x.astype(jnp.bfloat16)
    partials = _sc_spmm(xb, src, dst, edge_weight)
    scal = jnp.stack([conv_w[0, 0, 0, 0], conv_w[0, 0, 0, 1], conv_b[0]])
    return _combine(partials, x, scal)
